# baseline (speedup 1.0000x reference)
import jax
import jax.numpy as jnp
from jax import lax
from jax.experimental import pallas as pl
from jax.experimental.pallas import tpu as pltpu
from jax.experimental.pallas import tpu_sc as plsc

BATCH = 4
SEQ_LEN = 8192
FEAT_DIM = 768
NW = 32
ROWS_W = SEQ_LEN // NW
R = 16
NBLK = ROWS_W // R
LANES = 16
CGRP = FEAT_DIM // LANES
UNROLL = 8


def _body(in_hbm, pos_hbm, time_hbm, out_hbm, *scr):
    pbuf = scr[0]
    tbuf = scr[1]
    ibuf = scr[2:10]
    psem = scr[10]
    tsem = scr[11]
    isem = scr[12:16]
    osem = scr[16:24]

    wid = lax.axis_index("s") * 2 + lax.axis_index("c")
    base = wid * ROWS_W

    def wait_in(sem, vref):
        pltpu.make_async_copy(pos_hbm.at[pl.ds(0, R), :], vref, sem).wait()

    def wait_out(slot, b):
        pltpu.make_async_copy(
            ibuf[slot], out_hbm.at[b, pl.ds(0, R), :], osem[slot]
        ).wait()

    def vloop(body):
        @pl.loop(0, R)
        def _row(r):
            @pl.loop(0, CGRP, unroll=UNROLL)
            def _col(c):
                body(r, pl.ds(c * LANES, LANES))

    def do_block(k, p, prefetch, wait_prev_out):
        pn = 1 - p
        roff = base + k * R
        wait_in(psem, pbuf)
        wait_in(tsem, tbuf)

        def _combine(r, s):
            tbuf[r, s] = tbuf[r, s] + pbuf[r, s]

        vloop(_combine)

        if prefetch:
            roffn = roff + R
            pltpu.async_copy(pos_hbm.at[pl.ds(roffn, R), :], pbuf, psem)

        for b in range(BATCH):
            slot = 2 * b + p
            ib = ibuf[slot]
            wait_in(isem[b], ib)

            def _accum(r, s, ib=ib):
                plsc.addupdate(ib.at[r, s], tbuf[r, s])

            vloop(_accum)

            pltpu.async_copy(ib, out_hbm.at[b, pl.ds(roff, R), :], osem[slot])
            if prefetch:
                nslot = 2 * b + pn
                if wait_prev_out:
                    wait_out(nslot, b)
                pltpu.async_copy(
                    in_hbm.at[b, pl.ds(roff + R, R), :], ibuf[nslot], isem[b]
                )

        if prefetch:
            pltpu.async_copy(time_hbm.at[pl.ds(roff + R, R), :], tbuf, tsem)

    pltpu.async_copy(pos_hbm.at[pl.ds(base, R), :], pbuf, psem)
    pltpu.async_copy(time_hbm.at[pl.ds(base, R), :], tbuf, tsem)
    for b in range(BATCH):
        pltpu.async_copy(in_hbm.at[b, pl.ds(base, R), :], ibuf[2 * b], isem[b])

    do_block(0, 0, prefetch=True, wait_prev_out=False)

    @pl.loop(1, NBLK - 1, step=2)
    def _mid(k0):
        do_block(k0, 1, prefetch=True, wait_prev_out=True)
        do_block(k0 + 1, 0, prefetch=True, wait_prev_out=True)

    do_block(NBLK - 1, 1, prefetch=False, wait_prev_out=False)

    for b in range(BATCH):
        wait_out(2 * b, b)
        wait_out(2 * b + 1, b)


@jax.jit
def kernel(inputs, pos_table, time_table):
    mesh = plsc.VectorSubcoreMesh(core_axis_name="c", subcore_axis_name="s")
    return pl.kernel(
        _body,
        out_type=jax.ShapeDtypeStruct((BATCH, SEQ_LEN, FEAT_DIM), jnp.float32),
        mesh=mesh,
        compiler_params=pltpu.CompilerParams(use_tc_tiling_on_sc=True),
        scratch_types=(
            [pltpu.VMEM((R, FEAT_DIM), jnp.float32) for _ in range(10)]
            + [pltpu.SemaphoreType.DMA for _ in range(14)]
        ),
    )(inputs, pos_table, time_table)


# speedup vs baseline: 2.0709x; 2.0709x over previous
import jax
import jax.numpy as jnp
from jax import lax
from jax.experimental import pallas as pl
from jax.experimental.pallas import tpu as pltpu
from jax.experimental.pallas import tpu_sc as plsc

BATCH = 4
SEQ_LEN = 8192
FEAT_DIM = 768
NW = 32
ROWS_W = SEQ_LEN // NW
R = 8
NBLK = ROWS_W // R
LANES = 16
CGRP = FEAT_DIM // LANES
UNROLL = 8


def _body(in_hbm, pos_hbm, time_hbm, out_hbm, *scr):
    pbuf = scr[0:2]
    tbuf = scr[2:4]
    ibuf = scr[4:12]
    psem = scr[12:14]
    tsem = scr[14:16]
    isem = scr[16:24]
    osem = scr[24:32]

    wid = lax.axis_index("s") * 2 + lax.axis_index("c")
    base = wid * ROWS_W

    def wait_in(sem, vref):
        pltpu.make_async_copy(pos_hbm.at[pl.ds(0, R), :], vref, sem).wait()

    def wait_out(slot, b):
        pltpu.make_async_copy(
            ibuf[slot], out_hbm.at[b, pl.ds(0, R), :], osem[slot]
        ).wait()

    def vloop(body):
        @pl.loop(0, R)
        def _row(r):
            @pl.loop(0, CGRP, unroll=UNROLL)
            def _col(c):
                body(r, pl.ds(c * LANES, LANES))

    def do_block(k, p, prefetch, wait_prev_out):
        pn = 1 - p
        roff = base + k * R
        wait_in(psem[p], pbuf[p])
        wait_in(tsem[p], tbuf[p])
        if prefetch:
            roffn = roff + R
            pltpu.async_copy(pos_hbm.at[pl.ds(roffn, R), :], pbuf[pn], psem[pn])
            pltpu.async_copy(time_hbm.at[pl.ds(roffn, R), :], tbuf[pn], tsem[pn])
            for b in range(BATCH):
                nslot = 2 * b + pn
                if wait_prev_out:
                    wait_out(nslot, b)
                pltpu.async_copy(
                    in_hbm.at[b, pl.ds(roffn, R), :], ibuf[nslot], isem[nslot]
                )

        pb = pbuf[p]
        tb = tbuf[p]

        def _combine(r, s):
            pb[r, s] = pb[r, s] + tb[r, s]

        vloop(_combine)

        for b in range(BATCH):
            slot = 2 * b + p
            ib = ibuf[slot]
            wait_in(isem[slot], ib)

            def _accum(r, s, ib=ib):
                plsc.addupdate(ib.at[r, s], pb[r, s])

            vloop(_accum)

            pltpu.async_copy(ib, out_hbm.at[b, pl.ds(roff, R), :], osem[slot])

    pltpu.async_copy(pos_hbm.at[pl.ds(base, R), :], pbuf[0], psem[0])
    pltpu.async_copy(time_hbm.at[pl.ds(base, R), :], tbuf[0], tsem[0])
    for b in range(BATCH):
        pltpu.async_copy(
            in_hbm.at[b, pl.ds(base, R), :], ibuf[2 * b], isem[2 * b]
        )

    do_block(0, 0, prefetch=True, wait_prev_out=False)

    @pl.loop(1, NBLK - 1, step=2)
    def _mid(k0):
        do_block(k0, 1, prefetch=True, wait_prev_out=True)
        do_block(k0 + 1, 0, prefetch=True, wait_prev_out=True)

    do_block(NBLK - 1, 1, prefetch=False, wait_prev_out=False)

    for b in range(BATCH):
        wait_out(2 * b, b)
        wait_out(2 * b + 1, b)


@jax.jit
def kernel(inputs, pos_table, time_table):
    mesh = plsc.VectorSubcoreMesh(core_axis_name="c", subcore_axis_name="s")
    return pl.kernel(
        _body,
        out_type=jax.ShapeDtypeStruct((BATCH, SEQ_LEN, FEAT_DIM), jnp.float32),
        mesh=mesh,
        compiler_params=pltpu.CompilerParams(use_tc_tiling_on_sc=True),
        scratch_types=(
            [pltpu.VMEM((R, FEAT_DIM), jnp.float32) for _ in range(12)]
            + [pltpu.SemaphoreType.DMA for _ in range(20)]
        ),
    )(inputs, pos_table, time_table)


# speedup vs baseline: 2.1504x; 1.0383x over previous
import jax
import jax.numpy as jnp
from jax import lax
from jax.experimental import pallas as pl
from jax.experimental.pallas import tpu as pltpu
from jax.experimental.pallas import tpu_sc as plsc

BATCH = 4
SEQ_LEN = 8192
FEAT_DIM = 768
NW = 32
ROWS_W = SEQ_LEN // NW
R = 8
NBLK = ROWS_W // R
LANES = 16
CGRP = FEAT_DIM // LANES
UNROLL = 8
DEPTH = 3


def _body(in_hbm, pos_hbm, time_hbm, out_hbm, *scr):
    pbuf = scr[0:3]
    tbuf = scr[3:6]
    ibuf = scr[6:18]
    psem = scr[18:21]
    tsem = scr[21:24]
    isem = scr[24:36]
    osem = scr[36:48]

    wid = lax.axis_index("s") * 2 + lax.axis_index("c")
    base = wid * ROWS_W

    def wait_in(sem, vref):
        pltpu.make_async_copy(pos_hbm.at[pl.ds(0, R), :], vref, sem).wait()

    def wait_out(slot, b):
        pltpu.make_async_copy(
            ibuf[slot], out_hbm.at[b, pl.ds(0, R), :], osem[slot]
        ).wait()

    def vloop(body):
        @pl.loop(0, R)
        def _row(r):
            @pl.loop(0, CGRP, unroll=UNROLL)
            def _col(c):
                body(r, pl.ds(c * LANES, LANES))

    def do_block(k, p, prefetch, wait_prev_out):
        pn = (p + 1) % DEPTH
        roff = base + k * R
        wait_in(psem[p], pbuf[p])
        wait_in(tsem[p], tbuf[p])
        if prefetch:
            roffn = roff + R
            pltpu.async_copy(pos_hbm.at[pl.ds(roffn, R), :], pbuf[pn], psem[pn])
            pltpu.async_copy(time_hbm.at[pl.ds(roffn, R), :], tbuf[pn], tsem[pn])
            for b in range(BATCH):
                nslot = DEPTH * b + pn
                if wait_prev_out:
                    wait_out(nslot, b)
                pltpu.async_copy(
                    in_hbm.at[b, pl.ds(roffn, R), :], ibuf[nslot], isem[nslot]
                )

        pb = pbuf[p]
        tb = tbuf[p]

        def _combine(r, s):
            pb[r, s] = pb[r, s] + tb[r, s]

        vloop(_combine)

        for b in range(BATCH):
            slot = DEPTH * b + p
            ib = ibuf[slot]
            wait_in(isem[slot], ib)

            def _accum(r, s, ib=ib):
                plsc.addupdate(ib.at[r, s], pb[r, s])

            vloop(_accum)

            pltpu.async_copy(ib, out_hbm.at[b, pl.ds(roff, R), :], osem[slot])

    pltpu.async_copy(pos_hbm.at[pl.ds(base, R), :], pbuf[0], psem[0])
    pltpu.async_copy(time_hbm.at[pl.ds(base, R), :], tbuf[0], tsem[0])
    for b in range(BATCH):
        pltpu.async_copy(
            in_hbm.at[b, pl.ds(base, R), :], ibuf[DEPTH * b], isem[DEPTH * b]
        )

    do_block(0, 0, prefetch=True, wait_prev_out=False)
    do_block(1, 1, prefetch=True, wait_prev_out=False)

    @pl.loop(2, NBLK - 3, step=3)
    def _mid(k0):
        do_block(k0, 2, prefetch=True, wait_prev_out=True)
        do_block(k0 + 1, 0, prefetch=True, wait_prev_out=True)
        do_block(k0 + 2, 1, prefetch=True, wait_prev_out=True)

    do_block(NBLK - 3, 2, prefetch=True, wait_prev_out=True)
    do_block(NBLK - 2, 0, prefetch=True, wait_prev_out=True)
    do_block(NBLK - 1, 1, prefetch=False, wait_prev_out=False)

    for b in range(BATCH):
        for p in range(DEPTH):
            wait_out(DEPTH * b + p, b)


@jax.jit
def kernel(inputs, pos_table, time_table):
    mesh = plsc.VectorSubcoreMesh(core_axis_name="c", subcore_axis_name="s")
    return pl.kernel(
        _body,
        out_type=jax.ShapeDtypeStruct((BATCH, SEQ_LEN, FEAT_DIM), jnp.float32),
        mesh=mesh,
        compiler_params=pltpu.CompilerParams(use_tc_tiling_on_sc=True),
        scratch_types=(
            [pltpu.VMEM((R, FEAT_DIM), jnp.float32) for _ in range(18)]
            + [pltpu.SemaphoreType.DMA for _ in range(30)]
        ),
    )(inputs, pos_table, time_table)


# speedup vs baseline: 2.8572x; 1.3287x over previous
import jax
import jax.numpy as jnp
from jax import lax
from jax.experimental import pallas as pl
from jax.experimental.pallas import tpu as pltpu
from jax.experimental.pallas import tpu_sc as plsc

BATCH = 4
SEQ_LEN = 8192
FEAT_DIM = 768
NW = 32
ROWS_W = SEQ_LEN // NW
R = 8
NBLK = ROWS_W // R
LANES = 16
CGRP = FEAT_DIM // LANES
UNROLL = 8
DEPTH = 3


def _body(in_hbm, pos_hbm, time_hbm, out_hbm, *scr):
    pbuf = scr[0:3]
    tbuf = scr[3:6]
    ibuf = scr[6:18]
    psem = scr[18:21]
    tsem = scr[21:24]
    isem = scr[24:36]
    osem = scr[36:48]

    wid = lax.axis_index("s") * 2 + lax.axis_index("c")
    base = wid * ROWS_W

    def wait_in(sem, vref):
        pltpu.make_async_copy(pos_hbm.at[pl.ds(0, R), :], vref, sem).wait()

    def wait_out(slot, b):
        pltpu.make_async_copy(
            ibuf[slot], out_hbm.at[b, pl.ds(0, R), :], osem[slot]
        ).wait()

    def vloop(body):
        @pl.loop(0, R)
        def _row(r):
            @plsc.parallel_loop(0, CGRP, unroll=UNROLL)
            def _col(c):
                body(r, pl.ds(c * LANES, LANES))

    def do_block(k, p, prefetch, wait_prev_out):
        pn = (p + 1) % DEPTH
        roff = base + k * R
        wait_in(psem[p], pbuf[p])
        wait_in(tsem[p], tbuf[p])
        if prefetch:
            roffn = roff + R
            pltpu.async_copy(pos_hbm.at[pl.ds(roffn, R), :], pbuf[pn], psem[pn])
            pltpu.async_copy(time_hbm.at[pl.ds(roffn, R), :], tbuf[pn], tsem[pn])
            for b in range(BATCH):
                nslot = DEPTH * b + pn
                if wait_prev_out:
                    wait_out(nslot, b)
                pltpu.async_copy(
                    in_hbm.at[b, pl.ds(roffn, R), :], ibuf[nslot], isem[nslot]
                )

        pb = pbuf[p]
        tb = tbuf[p]

        for b in range(BATCH):
            slot = DEPTH * b + p
            ib = ibuf[slot]
            wait_in(isem[slot], ib)

            if b == 0:
                def _accum(r, s, ib=ib):
                    v = pb[r, s] + tb[r, s]
                    pb[r, s] = v
                    plsc.addupdate(ib.at[r, s], v)
            else:
                def _accum(r, s, ib=ib):
                    plsc.addupdate(ib.at[r, s], pb[r, s])

            vloop(_accum)

            pltpu.async_copy(ib, out_hbm.at[b, pl.ds(roff, R), :], osem[slot])

    pltpu.async_copy(pos_hbm.at[pl.ds(base, R), :], pbuf[0], psem[0])
    pltpu.async_copy(time_hbm.at[pl.ds(base, R), :], tbuf[0], tsem[0])
    for b in range(BATCH):
        pltpu.async_copy(
            in_hbm.at[b, pl.ds(base, R), :], ibuf[DEPTH * b], isem[DEPTH * b]
        )

    do_block(0, 0, prefetch=True, wait_prev_out=False)
    do_block(1, 1, prefetch=True, wait_prev_out=False)

    @pl.loop(2, NBLK - 3, step=3)
    def _mid(k0):
        do_block(k0, 2, prefetch=True, wait_prev_out=True)
        do_block(k0 + 1, 0, prefetch=True, wait_prev_out=True)
        do_block(k0 + 2, 1, prefetch=True, wait_prev_out=True)

    do_block(NBLK - 3, 2, prefetch=True, wait_prev_out=True)
    do_block(NBLK - 2, 0, prefetch=True, wait_prev_out=True)
    do_block(NBLK - 1, 1, prefetch=False, wait_prev_out=False)

    for b in range(BATCH):
        for p in range(DEPTH):
            wait_out(DEPTH * b + p, b)


@jax.jit
def kernel(inputs, pos_table, time_table):
    mesh = plsc.VectorSubcoreMesh(core_axis_name="c", subcore_axis_name="s")
    return pl.kernel(
        _body,
        out_type=jax.ShapeDtypeStruct((BATCH, SEQ_LEN, FEAT_DIM), jnp.float32),
        mesh=mesh,
        compiler_params=pltpu.CompilerParams(use_tc_tiling_on_sc=True),
        scratch_types=(
            [pltpu.VMEM((R, FEAT_DIM), jnp.float32) for _ in range(18)]
            + [pltpu.SemaphoreType.DMA for _ in range(30)]
        ),
    )(inputs, pos_table, time_table)
